# baseline (speedup 1.0000x reference)
import functools

import jax
import jax.numpy as jnp
from jax import lax
from jax.experimental import pallas as pl
from jax.experimental.pallas import tpu as pltpu
from jax.experimental.pallas import tpu_sc as plsc

DIM = 64
CH = 128

_info = plsc.get_sparse_core_info()
NC, NS = _info.num_cores, _info.num_subcores
NW = NC * NS


def _build(batch):
    assert batch % (NW * CH) == 0
    nch = batch // (NW * CH)
    mesh = plsc.VectorSubcoreMesh(core_axis_name="c", subcore_axis_name="s")

    @functools.partial(
        pl.kernel,
        mesh=mesh,
        out_type=jax.ShapeDtypeStruct((NW, nch, CH, DIM), jnp.float32),
        scratch_types=[
            pltpu.VMEM((nch, CH), jnp.int32),
            pltpu.VMEM((CH, DIM), jnp.float32),
            pltpu.SemaphoreType.DMA,
        ],
        compiler_params=pltpu.CompilerParams(use_tc_tiling_on_sc=False),
    )
    def k(tok_hbm, table_hbm, out_hbm, idx_v, rows_v, sem):
        wid = lax.axis_index("s") * NC + lax.axis_index("c")
        pltpu.sync_copy(tok_hbm.at[wid], idx_v)

        def chunk(c, carry):
            pltpu.async_copy(table_hbm.at[idx_v.at[c]], rows_v, sem).wait()
            pltpu.sync_copy(rows_v, out_hbm.at[wid, c])
            return carry

        lax.fori_loop(0, nch, chunk, 0)

    return k


def kernel(tokens, table):
    b, s = tokens.shape
    tok = tokens.reshape(-1).astype(jnp.int32)
    batch = b * s
    nch = batch // (NW * CH)
    out = _build(batch)(tok.reshape(NW, nch, CH), table)
    return out.reshape(b, s, DIM)


# speedup vs baseline: 1.0476x; 1.0476x over previous
import functools

import jax
import jax.numpy as jnp
from jax import lax
from jax.experimental import pallas as pl
from jax.experimental.pallas import tpu as pltpu
from jax.experimental.pallas import tpu_sc as plsc

DIM = 64
CH = 128

_info = plsc.get_sparse_core_info()
NC, NS = _info.num_cores, _info.num_subcores
NW = NC * NS


NB = 8
LAG = 4


def _build(batch):
    assert batch % (NW * CH) == 0
    nch = batch // (NW * CH)
    assert nch >= NB
    mesh = plsc.VectorSubcoreMesh(core_axis_name="c", subcore_axis_name="s")

    @functools.partial(
        pl.kernel,
        mesh=mesh,
        out_type=jax.ShapeDtypeStruct((NW, nch, CH, DIM), jnp.float32),
        scratch_types=[
            pltpu.VMEM((nch, CH), jnp.int32),
            pltpu.VMEM((NB, CH, DIM), jnp.float32),
            pltpu.SemaphoreType.DMA((NB,)),
            pltpu.SemaphoreType.DMA((NB,)),
        ],
        compiler_params=pltpu.CompilerParams(use_tc_tiling_on_sc=False),
    )
    def k(tok_hbm, table_hbm, out_hbm, idx_v, rows_v, gsem, ssem):
        wid = lax.axis_index("s") * NC + lax.axis_index("c")
        pltpu.sync_copy(tok_hbm.at[wid], idx_v)

        def step(i, carry):
            b = lax.rem(i, NB)

            @pl.when(i < nch)
            def _issue_gather():
                @pl.when(i >= NB)
                def _():
                    pltpu.make_async_copy(
                        rows_v.at[b], out_hbm.at[wid, i - NB], ssem.at[b]
                    ).wait()
                pltpu.async_copy(table_hbm.at[idx_v.at[i]], rows_v.at[b],
                                 gsem.at[b])

            j = i - LAG

            @pl.when((j >= 0) & (j < nch))
            def _issue_store():
                b2 = lax.rem(j, NB)
                pltpu.make_async_copy(
                    table_hbm.at[idx_v.at[j]], rows_v.at[b2], gsem.at[b2]
                ).wait()
                pltpu.async_copy(rows_v.at[b2], out_hbm.at[wid, j],
                                 ssem.at[b2])

            return carry

        lax.fori_loop(0, nch + LAG, step, 0)

        def drain(i, carry):
            j = nch - NB + i
            b = lax.rem(j, NB)
            pltpu.make_async_copy(rows_v.at[b], out_hbm.at[wid, j],
                                  ssem.at[b]).wait()
            return carry

        lax.fori_loop(0, NB, drain, 0)

    return k


def kernel(tokens, table):
    b, s = tokens.shape
    tok = tokens.reshape(-1).astype(jnp.int32)
    batch = b * s
    nch = batch // (NW * CH)
    out = _build(batch)(tok.reshape(NW, nch, CH), table)
    return out.reshape(b, s, DIM)


# speedup vs baseline: 1.0490x; 1.0013x over previous
import functools

import jax
import jax.numpy as jnp
from jax import lax
from jax.experimental import pallas as pl
from jax.experimental.pallas import tpu as pltpu
from jax.experimental.pallas import tpu_sc as plsc

DIM = 64
NB = 8
LAG = 4

_info = plsc.get_sparse_core_info()
NC, NS = _info.num_cores, _info.num_subcores
NW = NC * NS


def _build(b, s):
    rpc = 2
    tpc = rpc * s
    assert tpc <= 128 and b % (NW * rpc) == 0
    nch = b // (NW * rpc)
    assert nch >= NB
    mesh = plsc.VectorSubcoreMesh(core_axis_name="c", subcore_axis_name="s")

    @functools.partial(
        pl.kernel,
        mesh=mesh,
        out_type=jax.ShapeDtypeStruct((b, s, DIM), jnp.float32),
        scratch_types=[
            pltpu.VMEM((nch, tpc), jnp.int32),
            pltpu.VMEM((NB, tpc, DIM), jnp.float32),
            pltpu.SemaphoreType.DMA((NB,)),
            pltpu.SemaphoreType.DMA((NB,)),
        ],
        compiler_params=pltpu.CompilerParams(use_tc_tiling_on_sc=False),
    )
    def k(tok_hbm, table_hbm, out_hbm, idx_v, rows_v, gsem, ssem):
        wid = lax.axis_index("s") * NC + lax.axis_index("c")
        row0 = wid * (nch * rpc)
        pltpu.sync_copy(tok_hbm.at[wid], idx_v)

        def wait_store(c, buf):
            r = row0 + c * rpc
            for q in range(rpc):
                pltpu.make_async_copy(
                    rows_v.at[buf, pl.ds(q * s, s)], out_hbm.at[r + q],
                    ssem.at[buf],
                ).wait()

        def step(i, carry):
            buf = lax.rem(i, NB)

            @pl.when(i < nch)
            def _issue_gather():
                @pl.when(i >= NB)
                def _():
                    wait_store(i - NB, buf)
                pltpu.async_copy(table_hbm.at[idx_v.at[i]], rows_v.at[buf],
                                 gsem.at[buf])

            j = i - LAG

            @pl.when((j >= 0) & (j < nch))
            def _issue_store():
                b2 = lax.rem(j, NB)
                pltpu.make_async_copy(
                    table_hbm.at[idx_v.at[j]], rows_v.at[b2], gsem.at[b2]
                ).wait()
                r = row0 + j * rpc
                for q in range(rpc):
                    pltpu.async_copy(rows_v.at[b2, pl.ds(q * s, s)],
                                     out_hbm.at[r + q], ssem.at[b2])

            return carry

        lax.fori_loop(0, nch + LAG, step, 0)

        def drain(i, carry):
            j = nch - NB + i
            wait_store(j, lax.rem(j, NB))
            return carry

        lax.fori_loop(0, NB, drain, 0)

    return k


def kernel(tokens, table):
    b, s = tokens.shape
    rpc = 2
    nch = b // (NW * rpc)
    tok = tokens.reshape(-1).astype(jnp.int32).reshape(NW, nch, rpc * s)
    return _build(b, s)(tok, table)
